# Initial kernel scaffold; baseline (speedup 1.0000x reference)
import jax
import jax.numpy as jnp
from jax.experimental import pallas as pl


def kernel(src, src_lengths, embed, W_enc, W_dec, W_out):
    raise NotImplementedError("write your pallas kernel here")


# speedup vs baseline: 2.9404x; 2.9404x over previous
import jax
import jax.numpy as jnp
from jax.experimental import pallas as pl
from jax.experimental.pallas import tpu as pltpu

_V = 8000
_D = 512
_B = 8
_S = 128
_NB = 4
_T = 12
_DIV = 0.5
_LPAD = 16
_R = _NB * _B

_HIGH = jax.lax.Precision.HIGHEST
_DN = (((1,), (0,)), ((), ()))


def _fiota(shape, dim):
    return jax.lax.broadcasted_iota(jnp.int32, shape, dim).astype(jnp.float32)


def _dot_bf(a, b_bf):
    return jax.lax.dot_general(a.astype(jnp.bfloat16), b_bf, _DN,
                               preferred_element_type=jnp.float32)


def _decoder_body(src_ref, embed_ref, we_ref, wd_ref, wo_ref,
                  seq0_ref, sc0_ref, seq1_ref, sc1_ref):
    f32 = jnp.float32
    embed = embed_ref[...]
    wd = wd_ref[...]
    wo = wo_ref[...]

    enc = jnp.tanh(jax.lax.dot_general(src_ref[...], we_ref[...], _DN,
                                       preferred_element_type=f32))
    ctx = jnp.concatenate(
        [jnp.sum(enc[i * _S:(i + 1) * _S], axis=0, keepdims=True)
         for i in range(_B)], axis=0) * f32(1.0 / _S)
    ctx32 = jnp.concatenate([ctx] * _NB, axis=0)

    iota_v = _fiota((1, _V), 1)
    rowid = _fiota((_R, 1), 0)
    rowbeam = jnp.floor(rowid / _B)
    rowbatch = rowid - _B * rowbeam
    flatpos = rowbeam * _V + iota_v
    lane_t = jax.lax.broadcasted_iota(jnp.int32, (1, _LPAD), 1)
    colid = _fiota((1, _R), 1)
    colbeam = jnp.floor(colid / _B)
    colbatch = colid - _B * colbeam

    def decode_step(t, seq, gs, tokens, done, penalty):
        oh = (tokens == iota_v).astype(f32)
        emb = _dot_bf(oh, embed)
        h = jnp.tanh(_dot_bf(emb, wd) + ctx32)
        logits = _dot_bf(h, wo)
        mx = jnp.max(logits, axis=1, keepdims=True)
        sh = logits - mx
        ls = sh - jnp.log(jnp.sum(jnp.exp(sh), axis=1, keepdims=True))
        if penalty is not None:
            ls = ls + penalty
        st = gs + ls
        vals, beams, toks = [], [], []
        for _ in range(_NB):
            b0, b1, b2, b3 = st[0:8], st[8:16], st[16:24], st[24:32]
            rm = jnp.max(jnp.maximum(jnp.maximum(b0, b1),
                                     jnp.maximum(b2, b3)),
                         axis=1, keepdims=True)
            rm32 = jnp.concatenate([rm] * _NB, axis=0)
            cand = jnp.where(st == rm32, flatpos, f32(1e9))
            c0, c1, c2, c3 = cand[0:8], cand[8:16], cand[16:24], cand[24:32]
            idx = jnp.min(jnp.minimum(jnp.minimum(c0, c1),
                                      jnp.minimum(c2, c3)),
                          axis=1, keepdims=True)
            idx32 = jnp.concatenate([idx] * _NB, axis=0)
            st = jnp.where(flatpos == idx32, f32(-1e38), st)
            bm = jnp.floor(idx / _V)
            vals.append(rm)
            beams.append(bm)
            toks.append(idx - _V * bm)
        new_gs = jnp.concatenate(vals, axis=0)
        beamvec = jnp.concatenate(beams, axis=0)
        ntv = jnp.concatenate(toks, axis=0)
        perm = ((colbeam == beamvec) & (colbatch == rowbatch)).astype(f32)
        gseq = jax.lax.dot_general(perm, seq, _DN, precision=_HIGH)
        newseq = jnp.where(lane_t == t, ntv, gseq)
        live = done < f32(0.5)
        seq = jnp.where(live, newseq, seq)
        gs = jnp.where(live, new_gs, gs)
        tokens = jnp.where(live, ntv, jnp.zeros_like(ntv))
        allz = jnp.min(jnp.where(ntv == f32(0.0), f32(1.0), f32(0.0)),
                       axis=0, keepdims=True)
        done = jnp.maximum(done, allz)
        return seq, gs, tokens, done

    seq_init = jnp.broadcast_to(
        jnp.where(lane_t == 0, f32(1.0), f32(0.0)), (_R, _LPAD))
    gs_init = jnp.where(rowbeam == 0, f32(0.0), f32(-1e9))
    tok_init = jnp.ones((_R, 1), f32)
    done_init = jnp.zeros((1, 1), f32)

    def body0(t, carry):
        return decode_step(t, *carry, penalty=None)

    seq0, gs0, _, _ = jax.lax.fori_loop(
        1, _T, body0, (seq_init, gs_init, tok_init, done_init))
    seq0_ref[...] = seq0
    sc0_ref[...] = gs0

    def body1(t, carry):
        seq, gs, tokens, done, pres = carry
        colsel = (jax.lax.broadcasted_iota(jnp.int32, (_LPAD, 1), 0)
                  == (t - 1)).astype(f32)
        col = jax.lax.dot_general(seq0, colsel, _DN, precision=_HIGH)
        hit = (col == iota_v).astype(f32)
        hb = jnp.concatenate(
            [jnp.broadcast_to(
                jnp.max(hit[j * _B:(j + 1) * _B], axis=0, keepdims=True),
                (_B, _V)) for j in range(_NB)], axis=0)
        pres = jnp.maximum(pres, hb)
        seq, gs, tokens, done = decode_step(
            t, seq, gs, tokens, done, penalty=f32(-_DIV) * pres)
        return seq, gs, tokens, done, pres

    pres_init = jnp.zeros((_R, _V), f32)
    seq1, gs1, _, _, _ = jax.lax.fori_loop(
        1, _T, body1, (seq_init, gs_init, tok_init, done_init, pres_init))
    seq1_ref[...] = seq1
    sc1_ref[...] = gs1


def kernel(src, src_lengths, embed, W_enc, W_dec, W_out):
    del src_lengths
    bf16 = jnp.bfloat16
    f32 = jnp.float32
    src2 = src.reshape(_B * _S, _D).astype(bf16)
    out_shape = (
        jax.ShapeDtypeStruct((_R, _LPAD), f32),
        jax.ShapeDtypeStruct((_R, 1), f32),
        jax.ShapeDtypeStruct((_R, _LPAD), f32),
        jax.ShapeDtypeStruct((_R, 1), f32),
    )
    seq0, sc0, seq1, sc1 = pl.pallas_call(
        _decoder_body,
        out_shape=out_shape,
        compiler_params=pltpu.CompilerParams(
            vmem_limit_bytes=60 * 1024 * 1024),
    )(src2, embed.astype(bf16), W_enc.astype(bf16),
      W_dec.astype(bf16), W_out.astype(bf16))

    def mkseq(s):
        return (s[:, :_T].reshape(_NB, _B, _T).transpose(1, 0, 2)
                .astype(jnp.int32))

    def mksc(s):
        return s.reshape(_NB, _B).T

    final_sequences = jnp.concatenate([mkseq(seq0), mkseq(seq1)], axis=1)
    final_scores = jnp.concatenate([mksc(sc0), mksc(sc1)], axis=1)
    return final_sequences, final_scores


# speedup vs baseline: 3.2686x; 1.1116x over previous
import jax
import jax.numpy as jnp
from jax.experimental import pallas as pl
from jax.experimental.pallas import tpu as pltpu

_V = 8000
_D = 512
_B = 8
_S = 128
_NB = 4
_T = 12
_DIV = 0.5
_LPAD = 16
_R = _NB * _B

_HIGH = jax.lax.Precision.HIGHEST
_DN = (((1,), (0,)), ((), ()))


def _fiota(shape, dim):
    return jax.lax.broadcasted_iota(jnp.int32, shape, dim).astype(jnp.float32)


def _dot_bf(a, b_bf):
    return jax.lax.dot_general(a.astype(jnp.bfloat16), b_bf, _DN,
                               preferred_element_type=jnp.float32)


def _decoder_body(src_ref, embed_ref, we_ref, wd_ref, wo_ref,
                  seq0_ref, sc0_ref, seq1_ref, sc1_ref):
    f32 = jnp.float32
    embed = embed_ref[...]
    wd = wd_ref[...]
    wo = wo_ref[...]

    enc = jnp.tanh(jax.lax.dot_general(src_ref[...], we_ref[...], _DN,
                                       preferred_element_type=f32))
    ctx = jnp.concatenate(
        [jnp.sum(enc[i * _S:(i + 1) * _S], axis=0, keepdims=True)
         for i in range(_B)], axis=0) * f32(1.0 / _S)
    ctx32 = jnp.concatenate([ctx] * _NB, axis=0)

    iota_v = _fiota((1, _V), 1)
    iota_vb = _fiota((_B, _V), 1)
    rowid = _fiota((_R, 1), 0)
    rowbeam = jnp.floor(rowid / _B)
    rowbatch = rowid - _B * rowbeam
    lane_t = jax.lax.broadcasted_iota(jnp.int32, (1, _LPAD), 1)
    colid = _fiota((1, _R), 1)
    colbeam = jnp.floor(colid / _B)
    colbatch = colid - _B * colbeam

    def decode_step(t, seq, gs, tokens, done, pen):
        oh = (tokens == iota_v).astype(f32)
        emb = _dot_bf(oh, embed)
        h = jnp.tanh(_dot_bf(emb, wd) + ctx32)
        logits = _dot_bf(h, wo)
        mx = jnp.max(logits, axis=1, keepdims=True)
        sh = logits - mx
        ls = sh - jnp.log(jnp.sum(jnp.exp(sh), axis=1, keepdims=True))
        sts = []
        for j in range(_NB):
            lsj = ls[j * _B:(j + 1) * _B]
            if pen is not None:
                lsj = lsj + jnp.broadcast_to(pen[j], (_B, _V))
            sts.append(gs[j * _B:(j + 1) * _B] + lsj)
        vals, beams, toks = [], [], []
        for k in range(_NB):
            m01 = jnp.maximum(jnp.max(sts[0], axis=1, keepdims=True),
                              jnp.max(sts[1], axis=1, keepdims=True))
            m23 = jnp.maximum(jnp.max(sts[2], axis=1, keepdims=True),
                              jnp.max(sts[3], axis=1, keepdims=True))
            rm = jnp.maximum(m01, m23)
            idx = None
            for j in range(_NB):
                cj = jnp.where(sts[j] == rm, iota_vb, f32(1e9))
                lj = jnp.min(cj, axis=1, keepdims=True) + f32(j * _V)
                idx = lj if idx is None else jnp.minimum(idx, lj)
            bm = jnp.floor(idx / _V)
            tk = idx - _V * bm
            vals.append(rm)
            beams.append(bm)
            toks.append(tk)
            if k + 1 < _NB:
                for j in range(_NB):
                    hitj = (iota_vb + f32(j * _V)) == idx
                    sts[j] = jnp.where(hitj, f32(-1e38), sts[j])
        beamvec = jnp.concatenate(beams, axis=0)
        ntv = jnp.concatenate(toks, axis=0)
        perm = ((colbeam == beamvec) & (colbatch == rowbatch)).astype(f32)
        gseq = jax.lax.dot_general(perm, seq, _DN, precision=_HIGH)
        newseq = jnp.where(lane_t == t, ntv, gseq)
        live = done < f32(0.5)
        seq = jnp.where(live, newseq, seq)
        gs = jnp.where(live, jnp.concatenate(vals, axis=0), gs)
        tokens = jnp.where(live, ntv, jnp.zeros_like(ntv))
        allz = jnp.min(jnp.where(ntv == f32(0.0), f32(1.0), f32(0.0)),
                       axis=0, keepdims=True)
        done = jnp.maximum(done, allz)
        return seq, gs, tokens, done

    seq_init = jnp.broadcast_to(
        jnp.where(lane_t == 0, f32(1.0), f32(0.0)), (_R, _LPAD))
    gs_init = jnp.where(rowbeam == 0, f32(0.0), f32(-1e9))
    tok_init = jnp.ones((_R, 1), f32)
    done_init = jnp.zeros((1, 1), f32)

    def body0(t, carry):
        return decode_step(t, *carry, pen=None)

    seq0, gs0, _, _ = jax.lax.fori_loop(
        1, _T, body0, (seq_init, gs_init, tok_init, done_init))
    seq0_ref[...] = seq0
    sc0_ref[...] = gs0

    def body1(t, carry):
        seq, gs, tokens, done, pen = carry
        colsel = (jax.lax.broadcasted_iota(jnp.int32, (_LPAD, 1), 0)
                  == (t - 1)).astype(f32)
        col = jax.lax.dot_general(seq0, colsel, _DN, precision=_HIGH)
        pen = tuple(
            jnp.minimum(pen[j], jnp.min(
                jnp.where(col[j * _B:(j + 1) * _B] == iota_v,
                          f32(-_DIV), f32(0.0)),
                axis=0, keepdims=True))
            for j in range(_NB))
        seq, gs, tokens, done = decode_step(t, seq, gs, tokens, done, pen)
        return seq, gs, tokens, done, pen

    pen_init = tuple(jnp.zeros((1, _V), f32) for _ in range(_NB))
    seq1, gs1, _, _, _ = jax.lax.fori_loop(
        1, _T, body1, (seq_init, gs_init, tok_init, done_init, pen_init))
    seq1_ref[...] = seq1
    sc1_ref[...] = gs1


def kernel(src, src_lengths, embed, W_enc, W_dec, W_out):
    del src_lengths
    bf16 = jnp.bfloat16
    f32 = jnp.float32
    src2 = src.reshape(_B * _S, _D).astype(bf16)
    out_shape = (
        jax.ShapeDtypeStruct((_R, _LPAD), f32),
        jax.ShapeDtypeStruct((_R, 1), f32),
        jax.ShapeDtypeStruct((_R, _LPAD), f32),
        jax.ShapeDtypeStruct((_R, 1), f32),
    )
    seq0, sc0, seq1, sc1 = pl.pallas_call(
        _decoder_body,
        out_shape=out_shape,
        compiler_params=pltpu.CompilerParams(
            vmem_limit_bytes=60 * 1024 * 1024),
    )(src2, embed.astype(bf16), W_enc.astype(bf16),
      W_dec.astype(bf16), W_out.astype(bf16))

    def mkseq(s):
        return (s[:, :_T].reshape(_NB, _B, _T).transpose(1, 0, 2)
                .astype(jnp.int32))

    def mksc(s):
        return s.reshape(_NB, _B).T

    final_sequences = jnp.concatenate([mkseq(seq0), mkseq(seq1)], axis=1)
    final_scores = jnp.concatenate([mksc(sc0), mksc(sc1)], axis=1)
    return final_sequences, final_scores


# speedup vs baseline: 3.3272x; 1.0179x over previous
import jax
import jax.numpy as jnp
from jax.experimental import pallas as pl
from jax.experimental.pallas import tpu as pltpu

_V = 8000
_D = 512
_B = 8
_S = 128
_NB = 4
_T = 12
_DIV = 0.5
_LPAD = 16
_R = _NB * _B

_HIGH = jax.lax.Precision.HIGHEST
_DN = (((1,), (0,)), ((), ()))


def _fiota(shape, dim):
    return jax.lax.broadcasted_iota(jnp.int32, shape, dim).astype(jnp.float32)


def _dot_bf(a, b_bf):
    return jax.lax.dot_general(a.astype(jnp.bfloat16), b_bf, _DN,
                               preferred_element_type=jnp.float32)


def _decoder_body(src_ref, embed_ref, we_ref, wd_ref, wo_ref,
                  seq0_ref, sc0_ref, seq1_ref, sc1_ref):
    f32 = jnp.float32
    embed = embed_ref[...]
    wd = wd_ref[...]
    wo = wo_ref[...]

    enc = jnp.tanh(jax.lax.dot_general(src_ref[...], we_ref[...], _DN,
                                       preferred_element_type=f32))
    ctx = jnp.concatenate(
        [jnp.sum(enc[i * _S:(i + 1) * _S], axis=0, keepdims=True)
         for i in range(_B)], axis=0) * f32(1.0 / _S)
    ctx32 = jnp.concatenate([ctx] * _NB, axis=0)

    iota_v = _fiota((1, _V), 1)
    iota_vb = _fiota((_B, _V), 1)
    rowid = _fiota((_R, 1), 0)
    rowbeam = jnp.floor(rowid / _B)
    rowbatch = rowid - _B * rowbeam
    lane_t = jax.lax.broadcasted_iota(jnp.int32, (1, _LPAD), 1)
    colid = _fiota((1, _R), 1)
    colbeam = jnp.floor(colid / _B)
    colbatch = colid - _B * colbeam

    def decode_step(t, seq, gs, tokens, done, pen):
        oh = (tokens == iota_v).astype(f32)
        emb = _dot_bf(oh, embed)
        h = jnp.tanh(_dot_bf(emb, wd) + ctx32)
        logits = _dot_bf(h, wo)
        sts = []
        for j in range(_NB):
            lg = logits[j * _B:(j + 1) * _B]
            mxj = jnp.max(lg, axis=1, keepdims=True)
            shj = lg - mxj
            lsj = shj - jnp.log(jnp.sum(jnp.exp(shj), axis=1, keepdims=True))
            if pen is not None:
                lsj = lsj + jnp.broadcast_to(pen[j], (_B, _V))
            sts.append(gs[j * _B:(j + 1) * _B] + lsj)
        bvals, blanes = [], []
        for j in range(_NB):
            stj = sts[j]
            for r in range(_NB):
                mj = jnp.max(stj, axis=1, keepdims=True)
                lj = jnp.min(jnp.where(stj == mj, iota_vb, f32(1e9)),
                             axis=1, keepdims=True)
                bvals.append(mj)
                blanes.append(lj + f32(j * _V))
                if r + 1 < _NB:
                    stj = jnp.where(iota_vb == lj, f32(-1e38), stj)
        v16 = jnp.concatenate(bvals, axis=1)
        f16 = jnp.concatenate(blanes, axis=1)
        vals, beams, toks = [], [], []
        for k in range(_NB):
            best = jnp.max(v16, axis=1, keepdims=True)
            fl = jnp.min(jnp.where(v16 == best, f16, f32(1e9)),
                         axis=1, keepdims=True)
            if k + 1 < _NB:
                v16 = jnp.where(f16 == fl, f32(-1e38), v16)
            bm = jnp.floor(fl / _V)
            tk = fl - _V * bm
            vals.append(best)
            beams.append(bm)
            toks.append(tk)
        beamvec = jnp.concatenate(beams, axis=0)
        ntv = jnp.concatenate(toks, axis=0)
        perm = ((colbeam == beamvec) & (colbatch == rowbatch)).astype(f32)
        gseq = jax.lax.dot_general(perm, seq, _DN, precision=_HIGH)
        newseq = jnp.where(lane_t == t, ntv, gseq)
        live = done < f32(0.5)
        seq = jnp.where(live, newseq, seq)
        gs = jnp.where(live, jnp.concatenate(vals, axis=0), gs)
        tokens = jnp.where(live, ntv, jnp.zeros_like(ntv))
        allz = jnp.min(jnp.where(ntv == f32(0.0), f32(1.0), f32(0.0)),
                       axis=0, keepdims=True)
        done = jnp.maximum(done, allz)
        return seq, gs, tokens, done

    seq_init = jnp.broadcast_to(
        jnp.where(lane_t == 0, f32(1.0), f32(0.0)), (_R, _LPAD))
    gs_init = jnp.where(rowbeam == 0, f32(0.0), f32(-1e9))
    tok_init = jnp.ones((_R, 1), f32)
    done_init = jnp.zeros((1, 1), f32)

    def body0(t, carry):
        return decode_step(t, *carry, pen=None)

    seq0, gs0, _, _ = jax.lax.fori_loop(
        1, _T, body0, (seq_init, gs_init, tok_init, done_init))
    seq0_ref[...] = seq0
    sc0_ref[...] = gs0

    def body1(t, carry):
        seq, gs, tokens, done, pen = carry
        colsel = (jax.lax.broadcasted_iota(jnp.int32, (_LPAD, 1), 0)
                  == (t - 1)).astype(f32)
        col = jax.lax.dot_general(seq0, colsel, _DN, precision=_HIGH)
        pen = tuple(
            jnp.minimum(pen[j], jnp.min(
                jnp.where(col[j * _B:(j + 1) * _B] == iota_v,
                          f32(-_DIV), f32(0.0)),
                axis=0, keepdims=True))
            for j in range(_NB))
        seq, gs, tokens, done = decode_step(t, seq, gs, tokens, done, pen)
        return seq, gs, tokens, done, pen

    pen_init = tuple(jnp.zeros((1, _V), f32) for _ in range(_NB))
    seq1, gs1, _, _, _ = jax.lax.fori_loop(
        1, _T, body1, (seq_init, gs_init, tok_init, done_init, pen_init))
    seq1_ref[...] = seq1
    sc1_ref[...] = gs1


def kernel(src, src_lengths, embed, W_enc, W_dec, W_out):
    del src_lengths
    bf16 = jnp.bfloat16
    f32 = jnp.float32
    src2 = src.reshape(_B * _S, _D).astype(bf16)
    out_shape = (
        jax.ShapeDtypeStruct((_R, _LPAD), f32),
        jax.ShapeDtypeStruct((_R, 1), f32),
        jax.ShapeDtypeStruct((_R, _LPAD), f32),
        jax.ShapeDtypeStruct((_R, 1), f32),
    )
    seq0, sc0, seq1, sc1 = pl.pallas_call(
        _decoder_body,
        out_shape=out_shape,
        compiler_params=pltpu.CompilerParams(
            vmem_limit_bytes=60 * 1024 * 1024),
    )(src2, embed.astype(bf16), W_enc.astype(bf16),
      W_dec.astype(bf16), W_out.astype(bf16))

    def mkseq(s):
        return (s[:, :_T].reshape(_NB, _B, _T).transpose(1, 0, 2)
                .astype(jnp.int32))

    def mksc(s):
        return s.reshape(_NB, _B).T

    final_sequences = jnp.concatenate([mkseq(seq0), mkseq(seq1)], axis=1)
    final_scores = jnp.concatenate([mksc(sc0), mksc(sc1)], axis=1)
    return final_sequences, final_scores
